# Initial kernel scaffold; baseline (speedup 1.0000x reference)
import jax
import jax.numpy as jnp
from jax.experimental import pallas as pl


def kernel(coords, constraint_params, block_coord_offset, constraint_atoms, constraint_function_inds, block_pair_dispatch_indices):
    raise NotImplementedError("write your pallas kernel here")


# speedup vs baseline: 42.4062x; 42.4062x over previous
import functools

import jax
import jax.numpy as jnp
from jax import lax
from jax.experimental import pallas as pl
from jax.experimental.pallas import tpu as pltpu
from jax.experimental.pallas import tpu_sc as plsc

NCORES = 2
NSUB = 16
NWORK = NCORES * NSUB
L = 16

NP = 8
NB = 1250
NBB = NB * NB
MA = 30000
NC = 640000
ND = 200000

TABLE_R = 393216
TABLE = NWORK * TABLE_R
DUMPCELL = TABLE - 8

DISP_PAD = 200704

W = 20480
NCP = NWORK * W
CH = 1024
NCHUNK = W // CH
NR = CH // 128

ACC = 200192
DUMP = ND
SL = ACC // NSUB


def _mesh():
    return plsc.VectorSubcoreMesh(
        core_axis_name="c", subcore_axis_name="s",
        num_cores=NCORES, num_subcores=NSUB)


_SC_PARAMS = pltpu.CompilerParams(
    needs_layout_passes=False, use_tc_tiling_on_sc=False)


def _phase1_body(disp_hbm, bounds_hbm, lookup_hbm,
                 dchunk_v, tgt_v, val_v, zbuf_v, bounds_v):
    core = lax.axis_index("c")
    sub = lax.axis_index("s")
    wid = core * NSUB + sub
    iota = lax.iota(jnp.int32, L)

    def zb(i, _):
        zbuf_v[pl.ds(i * L, L)] = jnp.full((L,), -1, jnp.int32)
        return 0
    lax.fori_loop(0, 4096 // L, zb, 0)

    wbase = wid * TABLE_R
    def ms(i, _):
        pltpu.sync_copy(zbuf_v, lookup_hbm.at[pl.ds(wbase + i * 4096, 4096)])
        return 0
    lax.fori_loop(0, TABLE_R // 4096, ms, 0)

    pltpu.sync_copy(bounds_hbm, bounds_v)
    lo = jnp.max(plsc.load_gather(bounds_v, [jnp.full((L,), wid, jnp.int32)]))
    hi = jnp.max(plsc.load_gather(
        bounds_v, [jnp.full((L,), wid + 1, jnp.int32)]))
    k0 = lo // 128
    k1 = (hi + 127) // 128

    def sc(k, _):
        pltpu.sync_copy(disp_hbm.at[pl.ds(k * 128, 128)], dchunk_v)
        for j in range(8):
            cell = dchunk_v[pl.ds(j * L, L)]
            posn = k * 128 + j * L + iota
            valid = (posn >= lo) & (posn < hi)
            tgt_v[pl.ds(j * L, L)] = jnp.where(valid, cell, DUMPCELL)
            val_v[pl.ds(j * L, L)] = posn
        pltpu.sync_copy(val_v, lookup_hbm.at[tgt_v])
        return 0
    lax.fori_loop(k0, k1, sc, 0)


NG = 4
NPER = NSUB // NG
ZSL = NG * ACC // NSUB


def _phase2_body(cx_hbm, cy_hbm, cz_hbm, bco_hbm, pose0_hbm, pose3_hbm,
                 r0_hbm, a0_hbm, r3_hbm, a3_hbm, p0_hbm, p1_hbm, p2_hbm,
                 fn_hbm, lookup_hbm, out_hbm,
                 bco_v, pose0_v, pose3_v, r0_v, a0_v, r3_v, a3_v,
                 p0_v, p1_v, p2_v, fn_v,
                 row0_i, row3_i, c1_i, c2_i,
                 x0x_v, x0y_v, x0z_v, x3x_v, x3y_v, x3z_v,
                 pos1_v, pos2_v, sval_v, si1_i, si2_i,
                 tbuf_v, obuf_v, acc_sh, sem):
    core = lax.axis_index("c")
    sub = lax.axis_index("s")
    wid = core * NSUB + sub
    grp = sub // NPER
    parity = sub % NPER
    gbase = grp * ACC
    iota = lax.iota(jnp.int32, L)

    def zb(i, _):
        tbuf_v[pl.ds(i * L, L)] = jnp.zeros((L,), jnp.float32)
        return 0
    lax.fori_loop(0, 2048 // L, zb, 0)
    zbase = sub * ZSL
    nz = ZSL // 2048
    zt = ZSL - nz * 2048
    def za(i, _):
        pltpu.sync_copy(tbuf_v, acc_sh.at[pl.ds(zbase + i * 2048, 2048)])
        return 0
    lax.fori_loop(0, nz, za, 0)
    if zt:
        pltpu.sync_copy(tbuf_v.at[pl.ds(0, zt)],
                        acc_sh.at[pl.ds(zbase + nz * 2048, zt)])
    plsc.subcore_barrier()

    pltpu.sync_copy(bco_hbm, bco_v)

    def chunk(ch, _):
        base = wid * W + ch * CH
        pltpu.sync_copy(pose0_hbm.at[pl.ds(base, CH)], pose0_v)
        pltpu.sync_copy(pose3_hbm.at[pl.ds(base, CH)], pose3_v)
        pltpu.sync_copy(r0_hbm.at[pl.ds(base, CH)], r0_v)
        pltpu.sync_copy(a0_hbm.at[pl.ds(base, CH)], a0_v)
        pltpu.sync_copy(r3_hbm.at[pl.ds(base, CH)], r3_v)
        pltpu.sync_copy(a3_hbm.at[pl.ds(base, CH)], a3_v)
        pltpu.sync_copy(p0_hbm.at[pl.ds(base, CH)], p0_v)
        pltpu.sync_copy(p1_hbm.at[pl.ds(base, CH)], p1_v)
        pltpu.sync_copy(p2_hbm.at[pl.ds(base, CH)], p2_v)
        pltpu.sync_copy(fn_hbm.at[pl.ds(base, CH)], fn_v)

        def l1(r, _):
            for k in range(8):
                o = r * 128 + k * L
                pp0 = pose0_v[pl.ds(o, L)]
                pp3 = pose3_v[pl.ds(o, L)]
                rr0 = r0_v[pl.ds(o, L)]
                aa0 = a0_v[pl.ds(o, L)]
                rr3 = r3_v[pl.ds(o, L)]
                aa3 = a3_v[pl.ds(o, L)]
                off0 = plsc.load_gather(bco_v, [pp0 * NB + rr0])
                off3 = plsc.load_gather(bco_v, [pp3 * NB + rr3])
                row0_i[r, pl.ds(k * L, L)] = pp0 * MA + off0 + aa0
                row3_i[r, pl.ds(k * L, L)] = pp3 * MA + off3 + aa3
                c1_i[r, pl.ds(k * L, L)] = pp0 * NBB + rr0 * NB + rr3
                c2_i[r, pl.ds(k * L, L)] = pp0 * NBB + rr3 * NB + rr0
            return 0
        lax.fori_loop(0, NR, l1, 0)

        cps = []
        for k in range(NR):
            dsk = pl.ds(k * 128, 128)
            cps.append(pltpu.async_copy(
                cx_hbm.at[row0_i.at[k]], x0x_v.at[dsk], sem))
            cps.append(pltpu.async_copy(
                cy_hbm.at[row0_i.at[k]], x0y_v.at[dsk], sem))
            cps.append(pltpu.async_copy(
                cz_hbm.at[row0_i.at[k]], x0z_v.at[dsk], sem))
            cps.append(pltpu.async_copy(
                cx_hbm.at[row3_i.at[k]], x3x_v.at[dsk], sem))
            cps.append(pltpu.async_copy(
                cy_hbm.at[row3_i.at[k]], x3y_v.at[dsk], sem))
            cps.append(pltpu.async_copy(
                cz_hbm.at[row3_i.at[k]], x3z_v.at[dsk], sem))
            cps.append(pltpu.async_copy(
                lookup_hbm.at[c1_i.at[k]], pos1_v.at[dsk], sem))
            cps.append(pltpu.async_copy(
                lookup_hbm.at[c2_i.at[k]], pos2_v.at[dsk], sem))
        for c in cps:
            c.wait()

        def l2(r, _):
            for k in range(8):
                o = r * 128 + k * L
                dx = x0x_v[pl.ds(o, L)] - x3x_v[pl.ds(o, L)]
                dy = x0y_v[pl.ds(o, L)] - x3y_v[pl.ds(o, L)]
                dz = x0z_v[pl.ds(o, L)] - x3z_v[pl.ds(o, L)]
                d2 = dx * dx + dy * dy + dz * dz + 1e-12
                bits = lax.bitcast_convert_type(d2, jnp.int32)
                yb = jnp.int32(0x5F3759DF) - lax.shift_right_arithmetic(bits, 1)
                y = lax.bitcast_convert_type(yb, jnp.float32)
                y = y * (1.5 - 0.5 * d2 * y * y)
                y = y * (1.5 - 0.5 * d2 * y * y)
                y = y * (1.5 - 0.5 * d2 * y * y)
                d = d2 * y
                pp0 = p0_v[pl.ds(o, L)]
                pp1 = p1_v[pl.ds(o, L)]
                pp2 = p2_v[pl.ds(o, L)]
                fnv = fn_v[pl.ds(o, L)]
                t = (d - 5.0 * pp0) / (pp1 + 0.5)
                s0 = t * t
                lb = 2.0 * pp0
                ub = lb + 4.0 * pp2 + 1.0
                e1 = jnp.maximum(lb - d, 0.0)
                e2 = jnp.maximum(d - ub, 0.0)
                s1 = e1 * e1 + e2 * e2
                s = jnp.where(fnv == 0, s0, s1)
                pos1 = pos1_v[pl.ds(o, L)]
                pos2 = pos2_v[pl.ds(o, L)]
                rr0 = r0_v[pl.ds(o, L)]
                rr3 = r3_v[pl.ds(o, L)]
                gid = base + o + iota
                real = gid < NC
                v1 = (pos1 >= 0) & real
                v2 = (pos2 >= 0) & (rr0 != rr3) & real
                si1_i[r, pl.ds(k * L, L)] = gbase + jnp.where(v1, pos1, DUMP)
                si2_i[r, pl.ds(k * L, L)] = gbase + jnp.where(v2, pos2, DUMP)
                sval_v[r, pl.ds(k * L, L)] = s
            return 0
        lax.fori_loop(0, NR, l2, 0)

        for p in range(NPER):
            plsc.subcore_barrier()
            @pl.when(parity == p)
            def _():
                for k in range(NR):
                    pltpu.sync_copy(sval_v.at[k], acc_sh.at[si1_i.at[k]],
                                    add=True)
                    pltpu.sync_copy(sval_v.at[k], acc_sh.at[si2_i.at[k]],
                                    add=True)
        return 0

    lax.fori_loop(0, NCHUNK, chunk, 0)

    plsc.subcore_barrier()

    obase = sub * SL
    hbase = core * ACC + obase
    nblk = SL // 2048
    tail = SL - nblk * 2048

    def red_block(off, size):
        def zc(i, _):
            obuf_v[pl.ds(i * L, L)] = jnp.zeros((L,), jnp.float32)
            return 0
        lax.fori_loop(0, size // L, zc, 0)
        def rg(g, _):
            pltpu.sync_copy(
                acc_sh.at[pl.ds(g * ACC + obase + off, size)],
                tbuf_v.at[pl.ds(0, size)])
            def av(i, _):
                obuf_v[pl.ds(i * L, L)] = (obuf_v[pl.ds(i * L, L)]
                                           + tbuf_v[pl.ds(i * L, L)])
                return 0
            lax.fori_loop(0, size // L, av, 0)
            return 0
        lax.fori_loop(0, NG, rg, 0)
        pltpu.sync_copy(obuf_v.at[pl.ds(0, size)],
                        out_hbm.at[pl.ds(hbase + off, size)])

    def wo(i, _):
        red_block(i * 2048, 2048)
        return 0
    lax.fori_loop(0, nblk, wo, 0)
    if tail:
        red_block(nblk * 2048, tail)


def _add_body(a_ref, o_ref):
    o_ref[...] = a_ref[0] + a_ref[1]


def kernel(coords, constraint_params, block_coord_offset, constraint_atoms,
           constraint_function_inds, block_pair_dispatch_indices):
    assert coords.shape == (NP, MA, 3)
    assert constraint_atoms.shape == (NC, 4, 3)
    assert block_pair_dispatch_indices.shape == (3, ND)
    assert block_coord_offset.shape == (NP, NB)

    cf = coords.reshape(NP * MA, 3)
    cx = cf[:, 0]
    cy = cf[:, 1]
    cz = cf[:, 2]
    bco = block_coord_offset.reshape(-1).astype(jnp.int32)

    pose0 = constraint_atoms[:, 0, 0]
    pose3 = constraint_atoms[:, 3, 0]
    r0 = constraint_atoms[:, 0, 1]
    a0 = constraint_atoms[:, 0, 2]
    r3 = constraint_atoms[:, 3, 1]
    a3 = constraint_atoms[:, 3, 2]
    padc = NCP - NC
    pads = lambda x: jnp.pad(x, (0, padc))
    pose0, pose3, r0, a0, r3, a3 = map(pads, (pose0, pose3, r0, a0, r3, a3))
    fn = pads(constraint_function_inds)
    p0 = pads(constraint_params[:, 0])
    p1 = pads(constraint_params[:, 1])
    p2 = pads(constraint_params[:, 2])

    d0 = block_pair_dispatch_indices[0]
    d1 = block_pair_dispatch_indices[1]
    d2 = block_pair_dispatch_indices[2]
    dflat = d0 * NBB + d1 * NB + d2
    dflat_pad = jnp.pad(dflat, (0, DISP_PAD - ND))
    bounds = jnp.searchsorted(
        dflat, jnp.arange(NWORK + 1, dtype=jnp.int32) * TABLE_R
    ).astype(jnp.int32)
    bounds = jnp.pad(bounds, (0, 40 - (NWORK + 1)), constant_values=ND)

    lookup = pl.kernel(
        _phase1_body,
        out_type=jax.ShapeDtypeStruct((TABLE,), jnp.int32),
        mesh=_mesh(),
        compiler_params=_SC_PARAMS,
        scratch_types=[
            pltpu.VMEM((128,), jnp.int32),
            pltpu.VMEM((128,), jnp.int32),
            pltpu.VMEM((128,), jnp.int32),
            pltpu.VMEM((4096,), jnp.int32),
            pltpu.VMEM((40,), jnp.int32),
        ],
    )(dflat_pad, bounds)

    partials = pl.kernel(
        _phase2_body,
        out_type=jax.ShapeDtypeStruct((NCORES * ACC,), jnp.float32),
        mesh=_mesh(),
        compiler_params=_SC_PARAMS,
        scratch_types=[
            pltpu.VMEM((NP * NB,), jnp.int32),
            pltpu.VMEM((CH,), jnp.int32),
            pltpu.VMEM((CH,), jnp.int32),
            pltpu.VMEM((CH,), jnp.int32),
            pltpu.VMEM((CH,), jnp.int32),
            pltpu.VMEM((CH,), jnp.int32),
            pltpu.VMEM((CH,), jnp.int32),
            pltpu.VMEM((CH,), jnp.float32),
            pltpu.VMEM((CH,), jnp.float32),
            pltpu.VMEM((CH,), jnp.float32),
            pltpu.VMEM((CH,), jnp.int32),
            pltpu.VMEM((NR, 128), jnp.int32),
            pltpu.VMEM((NR, 128), jnp.int32),
            pltpu.VMEM((NR, 128), jnp.int32),
            pltpu.VMEM((NR, 128), jnp.int32),
            pltpu.VMEM((CH,), jnp.float32),
            pltpu.VMEM((CH,), jnp.float32),
            pltpu.VMEM((CH,), jnp.float32),
            pltpu.VMEM((CH,), jnp.float32),
            pltpu.VMEM((CH,), jnp.float32),
            pltpu.VMEM((CH,), jnp.float32),
            pltpu.VMEM((CH,), jnp.int32),
            pltpu.VMEM((CH,), jnp.int32),
            pltpu.VMEM((NR, 128), jnp.float32),
            pltpu.VMEM((NR, 128), jnp.int32),
            pltpu.VMEM((NR, 128), jnp.int32),
            pltpu.VMEM((2048,), jnp.float32),
            pltpu.VMEM((2048,), jnp.float32),
            pltpu.VMEM_SHARED((NG * ACC,), jnp.float32),
            pltpu.SemaphoreType.DMA,
        ],
    )(cx, cy, cz, bco, pose0, pose3, r0, a0, r3, a3, p0, p1, p2, fn, lookup)

    summed = pl.pallas_call(
        _add_body,
        out_shape=jax.ShapeDtypeStruct((ACC // 128, 128), jnp.float32),
    )(partials.reshape(NCORES, ACC // 128, 128))

    return summed.reshape(-1)[:ND]
